# baseline (speedup 1.0000x reference)
import jax
import jax.numpy as jnp
from jax import lax
from jax.experimental import pallas as pl
from jax.experimental.pallas import tpu as pltpu
from jax.experimental.pallas import tpu_sc as plsc

NUM_USERS = 25000
NUM_ITEMS = 75000
NUM_LAYERS = 3
D = 32
B = 128
NCORES = 2
NSUB = 16
G = 12
CB = G * B


def _layer_body(nb, h, h16, pt, ego_hbm, src_hbm, dst_hbm, val_hbm, zeros_hbm,
                out_hbm, srcc, dstc, valc, rows, idxb, acc, semg, sems, semi):
    c = lax.axis_index("c")
    s = lax.axis_index("s")
    base_row = c * h

    pltpu.sync_copy(zeros_hbm, acc.at[pl.ds(s * h16, h16)])
    plsc.subcore_barrier()

    tile_e0 = s * pt
    nchunk = nb // G

    def idx_start(ci, bufset):
        e0 = tile_e0 + ci * CB
        pltpu.async_copy(src_hbm.at[pl.ds(e0, CB)], srcc[bufset], semi)
        pltpu.async_copy(dst_hbm.at[pl.ds(e0, CB)], dstc[bufset], semi)
        pltpu.async_copy(val_hbm.at[pl.ds(e0, CB)], valc[bufset], semi)

    def idx_wait(ci, bufset):
        e0 = tile_e0 + ci * CB
        pltpu.make_async_copy(src_hbm.at[pl.ds(e0, CB)], srcc[bufset],
                              semi).wait()
        pltpu.make_async_copy(dst_hbm.at[pl.ds(e0, CB)], dstc[bufset],
                              semi).wait()
        pltpu.make_async_copy(val_hbm.at[pl.ds(e0, CB)], valc[bufset],
                              semi).wait()

    def gather_start(j, slot, bufset):
        pltpu.async_copy(ego_hbm.at[srcc[bufset].at[pl.ds(j * B, B)]],
                         rows[slot], semg[slot])

    def gather_wait(j, slot, bufset):
        pltpu.make_async_copy(ego_hbm.at[srcc[bufset].at[pl.ds(j * B, B)]],
                              rows[slot], semg[slot]).wait()

    def scatter_start(slot):
        pltpu.async_copy(rows[slot], acc.at[idxb[slot]], sems[slot], add=True)

    def scatter_wait(slot):
        pltpu.make_async_copy(rows[slot], acc.at[idxb[slot]],
                              sems[slot]).wait()

    def compute_block(j, slot, bufset):
        def body16(k, carry):
            dv = dstc[bufset][pl.ds(j * B + k * 16, 16)]
            lv = dv - base_row
            ok = (lv >= 0) & (lv < h)
            sc = jnp.where(ok, valc[bufset][pl.ds(j * B + k * 16, 16)], 0.0)
            lc = jnp.minimum(jnp.maximum(lv, 0), h - 1)
            idxb[slot][pl.ds(k * 16, 16)] = lc
            for u in range(16):
                e = k * 16 + u
                sv = sc[u]
                rows[slot][e, pl.ds(0, 16)] = rows[slot][e, pl.ds(0, 16)] * sv
                rows[slot][e, pl.ds(16, 16)] = rows[slot][e, pl.ds(16, 16)] * sv
            return carry

        lax.fori_loop(0, B // 16, body16, 0)

    def run_chunk(ci, bufset, prefetch):
        idx_wait(ci, bufset)
        if prefetch:
            idx_start(ci + 1, 1 - bufset)

        gather_start(0, 0, bufset)
        gather_start(1, 1, bufset)
        gather_wait(0, 0, bufset)
        compute_block(0, 0, bufset)
        scatter_start(0)
        gather_start(2, 2, bufset)
        gather_wait(1, 1, bufset)
        compute_block(1, 1, bufset)
        scatter_start(1)

        def triple_body(p, carry):
            j0 = 3 * p + 2
            for (dj, slot) in ((0, 2), (1, 0), (2, 1)):
                j = j0 + dj
                nslot = (slot + 1) % 3
                scatter_wait(nslot)
                gather_start(j + 1, nslot, bufset)
                gather_wait(j, slot, bufset)
                compute_block(j, slot, bufset)
                scatter_start(slot)
            return carry

        lax.fori_loop(0, (G - 3) // 3, triple_body, 0)

        gather_wait(G - 1, 2, bufset)
        compute_block(G - 1, 2, bufset)
        scatter_start(2)
        scatter_wait(0)
        scatter_wait(1)
        scatter_wait(2)

    idx_start(0, 0)

    def pair_body(p, carry):
        run_chunk(2 * p, 0, True)
        run_chunk(2 * p + 1, 1, True)
        return carry

    lax.fori_loop(0, nchunk // 2 - 1, pair_body, 0)
    run_chunk(nchunk - 2, 0, True)
    run_chunk(nchunk - 1, 1, False)

    plsc.subcore_barrier()
    pltpu.sync_copy(acc.at[pl.ds(s * h16, h16)],
                    out_hbm.at[pl.ds(base_row + s * h16, h16)])


def _make_layer(n, e_pad):
    h = n // NCORES
    h16 = h // NSUB
    pt = e_pad // NSUB
    nb = pt // B
    mesh = plsc.VectorSubcoreMesh(core_axis_name="c", subcore_axis_name="s")

    def body(ego_hbm, src_hbm, dst_hbm, val_hbm, zeros_hbm, out_hbm,
             srcc0, srcc1, dstc0, dstc1, valc0, valc1,
             rows0, rows1, rows2, idxb0, idxb1, idxb2, acc,
             semg0, semg1, semg2, sems0, sems1, sems2, semi):
        _layer_body(nb, h, h16, pt, ego_hbm, src_hbm, dst_hbm, val_hbm,
                    zeros_hbm, out_hbm, (srcc0, srcc1), (dstc0, dstc1),
                    (valc0, valc1), (rows0, rows1, rows2),
                    (idxb0, idxb1, idxb2), acc,
                    (semg0, semg1, semg2), (sems0, sems1, sems2), semi)

    return pl.kernel(
        body,
        out_type=jax.ShapeDtypeStruct((n, D), jnp.float32),
        mesh=mesh,
        scratch_types=[
            pltpu.VMEM((CB,), jnp.int32),
            pltpu.VMEM((CB,), jnp.int32),
            pltpu.VMEM((CB,), jnp.int32),
            pltpu.VMEM((CB,), jnp.int32),
            pltpu.VMEM((CB,), jnp.float32),
            pltpu.VMEM((CB,), jnp.float32),
            pltpu.VMEM((B, D), jnp.float32),
            pltpu.VMEM((B, D), jnp.float32),
            pltpu.VMEM((B, D), jnp.float32),
            pltpu.VMEM((B,), jnp.int32),
            pltpu.VMEM((B,), jnp.int32),
            pltpu.VMEM((B,), jnp.int32),
            pltpu.VMEM_SHARED((h, D), jnp.float32),
            pltpu.SemaphoreType.DMA,
            pltpu.SemaphoreType.DMA,
            pltpu.SemaphoreType.DMA,
            pltpu.SemaphoreType.DMA,
            pltpu.SemaphoreType.DMA,
            pltpu.SemaphoreType.DMA,
            pltpu.SemaphoreType.DMA,
        ],
        compiler_params=pltpu.CompilerParams(use_tc_tiling_on_sc=False),
        name="lightgcn_spmm_layer",
    )


def _mean_body(e0, e1, e2, e3, out):
    out[...] = (e0[...] + e1[...] + e2[...] + e3[...]) * 0.25


def _mean4(egos, n):
    rows = n * D // 128
    blk = 1000
    grid = rows // blk
    flat = [e.reshape(rows, 128) for e in egos]
    spec = pl.BlockSpec((blk, 128), lambda i: (i, 0))
    out = pl.pallas_call(
        _mean_body,
        out_shape=jax.ShapeDtypeStruct((rows, 128), jnp.float32),
        grid=(grid,),
        in_specs=[spec] * 4,
        out_specs=spec,
    )(*flat)
    return out.reshape(n, D)


def kernel(adj_indices, adj_values, user_emb, item_emb):
    n = user_emb.shape[0] + item_emb.shape[0]
    row_chunk = NCORES * NSUB * 8
    n_pad = ((n + row_chunk - 1) // row_chunk) * row_chunk
    e = adj_values.shape[0]
    chunk = NSUB * B * G * 2
    e_pad = ((e + chunk - 1) // chunk) * chunk

    dst = adj_indices[0]
    src = adj_indices[1]
    pad = e_pad - e
    if pad:
        dst = jnp.pad(dst, (0, pad))
        src = jnp.pad(src, (0, pad))
        val = jnp.pad(adj_values, (0, pad))
    else:
        val = adj_values
    zeros = jnp.zeros((n_pad // NCORES // NSUB, D), jnp.float32)

    ego0 = jnp.concatenate(
        [user_emb, item_emb,
         jnp.zeros((n_pad - n, D), jnp.float32)], axis=0)
    layer = _make_layer(n_pad, e_pad)
    ego1 = layer(ego0, src, dst, val, zeros)
    ego2 = layer(ego1, src, dst, val, zeros)
    ego3 = layer(ego2, src, dst, val, zeros)

    final = _mean4([x[:n] for x in (ego0, ego1, ego2, ego3)], n)
    nu = user_emb.shape[0]
    return (final[:nu], final[nu:])
